# baseline (speedup 1.0000x reference)
import functools

import jax
import jax.numpy as jnp
from jax import lax
from jax.experimental import pallas as pl
from jax.experimental.pallas import tpu as pltpu
from jax.experimental.pallas import tpu_sc as plsc

N = 10000
E = 160000
WE = 32000
L = 128
S = 15

NC = 2
NS = 16
NW = NC * NS


def _ln_rows(y, g, bb):
    m = jnp.mean(y, axis=-1, keepdims=True)
    v = jnp.mean((y - m) ** 2, axis=-1, keepdims=True)
    return (y - m) * lax.rsqrt(v + 1e-5) * g + bb


def _enc_body(x_ref, w1_ref, b1_ref, w2_ref, b2_ref, g_ref, bb_ref, o_ref):
    x = x_ref[...]
    h = jnp.maximum(
        jnp.dot(x, w1_ref[...], preferred_element_type=jnp.float32) + b1_ref[...], 0.0)
    y = jnp.dot(h, w2_ref[...], preferred_element_type=jnp.float32) + b2_ref[...]
    o_ref[...] = _ln_rows(y, g_ref[...], bb_ref[...])


def _enc_mlp(x, w1, b1, w2, b2, g, bb, br):
    r, k = x.shape
    full = lambda *s: pl.BlockSpec(s, lambda i: tuple(0 for _ in s))
    return pl.pallas_call(
        _enc_body,
        grid=(r // br,),
        in_specs=[
            pl.BlockSpec((br, k), lambda i: (i, 0)),
            full(k, L), full(1, L), full(L, L), full(1, L), full(1, L), full(1, L),
        ],
        out_specs=pl.BlockSpec((br, L), lambda i: (i, 0)),
        out_shape=jax.ShapeDtypeStruct((r, L), jnp.float32),
    )(x, w1, b1, w2, b2, g, bb)


def _pack_bf16_pair(lo, hi):
    ulo = lax.bitcast_convert_type(lo, jnp.uint32)
    uhi = lax.bitcast_convert_type(hi, jnp.uint32)
    rlo = (ulo + jnp.uint32(0x7FFF) + ((ulo >> 16) & jnp.uint32(1))) >> 16
    rhi = ((uhi + jnp.uint32(0x7FFF) + ((uhi >> 16) & jnp.uint32(1)))
           & jnp.uint32(0xFFFF0000))
    return rlo | rhi


def _unpack_bf16_pair(u):
    lo = lax.bitcast_convert_type(u << 16, jnp.float32)
    hi = lax.bitcast_convert_type(u & jnp.uint32(0xFFFF0000), jnp.float32)
    return jnp.concatenate([lo, hi], axis=-1)


def _proj_body(x_ref, w_ref, o_ref):
    o_ref[...] = jnp.dot(x_ref[...], w_ref[0],
                         preferred_element_type=jnp.float32)


def _proj(node, wstack, bn):
    nb = N // bn
    return pl.pallas_call(
        _proj_body,
        grid=(4, nb),
        in_specs=[
            pl.BlockSpec((bn, L), lambda t, i: (i, 0)),
            pl.BlockSpec((1, L, L), lambda t, i: (t, 0, 0)),
        ],
        out_specs=pl.BlockSpec((bn, L), lambda t, i: (t * nb + i, 0)),
        out_shape=jax.ShapeDtypeStruct((4 * N, L), jnp.float32),
    )(node, wstack)


def _mm_bias_body(x_ref, w_ref, b_ref, o_ref):
    o_ref[...] = jnp.dot(x_ref[...], w_ref[...],
                         preferred_element_type=jnp.float32) + b_ref[...]


def _mm_bias(x, w, b, br):
    r = x.shape[0]
    full = lambda *s: pl.BlockSpec(s, lambda i: tuple(0 for _ in s))
    return pl.pallas_call(
        _mm_bias_body,
        grid=(r // br,),
        in_specs=[pl.BlockSpec((br, L), lambda i: (i, 0)),
                  full(L, L), full(1, L)],
        out_specs=pl.BlockSpec((br, L), lambda i: (i, 0)),
        out_shape=jax.ShapeDtypeStruct((r, L), jnp.float32),
    )(x, w, b)


def _edge_body(x_ref, t_ref, g12_ref, w2_ref, b2_ref,
               g_ref, bb_ref, o_ref):
    x = x_ref[...]
    h = jnp.maximum(t_ref[...] + g12_ref[...], 0.0)
    y = jnp.dot(h, w2_ref[...],
                preferred_element_type=jnp.float32) + b2_ref[...]
    o_ref[...] = x + _ln_rows(y, g_ref[...], bb_ref[...])


def _edge_mlp(x, t, gsum, off, w2, b2, g, bb, be):
    r = x.shape[0]
    full = lambda *s: pl.BlockSpec(s, lambda i: tuple(0 for _ in s))
    o1 = off // be
    return pl.pallas_call(
        _edge_body,
        grid=(r // be,),
        in_specs=[
            pl.BlockSpec((be, L), lambda i: (i, 0)),
            pl.BlockSpec((be, L), lambda i: (i, 0)),
            pl.BlockSpec((be, L), lambda i, o=o1: (o + i, 0)),
            full(L, L), full(1, L), full(1, L), full(1, L),
        ],
        out_specs=pl.BlockSpec((be, L), lambda i: (i, 0)),
        out_shape=jax.ShapeDtypeStruct((r, L), jnp.float32),
    )(x, t, gsum, w2, b2, g, bb)


def _node_body(x_ref, t_ref, m0_ref, m1_ref, w0_ref, w1_ref, pm_ref, pw_ref,
               w2_ref, b2_ref, g_ref, bb_ref, o_ref):
    x = x_ref[...]
    mg = m0_ref[0] + m1_ref[0]
    wg = w0_ref[0] + w1_ref[0]
    t = (t_ref[...]
         + jnp.dot(mg, pm_ref[...], preferred_element_type=jnp.float32)
         + jnp.dot(wg, pw_ref[...], preferred_element_type=jnp.float32))
    h = jnp.maximum(t, 0.0)
    y = jnp.dot(h, w2_ref[...],
                preferred_element_type=jnp.float32) + b2_ref[...]
    o_ref[...] = x + _ln_rows(y, g_ref[...], bb_ref[...])


def _node_mlp(node, tnd, magg, wagg, pm, pw, w2, b2, g, bb, bn):
    full = lambda *s: pl.BlockSpec(s, lambda i: tuple(0 for _ in s))
    part = lambda c: pl.BlockSpec((1, bn, L), lambda i, c=c: (c, i, 0))
    return pl.pallas_call(
        _node_body,
        grid=(N // bn,),
        in_specs=[
            pl.BlockSpec((bn, L), lambda i: (i, 0)),
            pl.BlockSpec((bn, L), lambda i: (i, 0)),
            part(0), part(1), part(0), part(1),
            full(L, L), full(L, L),
            full(L, L), full(1, L), full(1, L), full(1, L),
        ],
        out_specs=pl.BlockSpec((bn, L), lambda i: (i, 0)),
        out_shape=jax.ShapeDtypeStruct((N, L), jnp.float32),
    )(node, tnd, magg, magg, wagg, wagg, pm, pw, w2, b2, g, bb)


def _dec_body(x_ref, w1_ref, b1_ref, w2_ref, b2_ref, o_ref):
    h = jnp.maximum(
        jnp.dot(x_ref[...], w1_ref[...], preferred_element_type=jnp.float32)
        + b1_ref[...], 0.0)
    o_ref[...] = jnp.dot(h, w2_ref[...],
                         preferred_element_type=jnp.float32) + b2_ref[...]


def _decoder(node, w1, b1, w2p, b2p, bn):
    full = lambda *s: pl.BlockSpec(s, lambda i: tuple(0 for _ in s))
    return pl.pallas_call(
        _dec_body,
        grid=(N // bn,),
        in_specs=[pl.BlockSpec((bn, L), lambda i: (i, 0)),
                  full(L, L), full(1, L), full(L, L), full(1, L)],
        out_specs=pl.BlockSpec((bn, L), lambda i: (i, 0)),
        out_shape=jax.ShapeDtypeStruct((N, L), jnp.float32),
    )(node, w1, b1, w2p, b2p)


def _sc_mesh():
    return plsc.VectorSubcoreMesh(core_axis_name="c", subcore_axis_name="s",
                                  num_cores=NC, num_subcores=NS)

_G_CH = 200


def _sc_gather_add(table, idx1, idx2):
    r = idx1.shape[0]
    per_w = r // NW
    nch = per_w // _G_CH
    assert nch % 2 == 0

    @functools.partial(
        pl.kernel,
        out_type=jax.ShapeDtypeStruct((r, L), jnp.float32),
        mesh=_sc_mesh(),
        scratch_types=[
            pltpu.VMEM((per_w,), jnp.int32),
            pltpu.VMEM((per_w,), jnp.int32),
            pltpu.VMEM((_G_CH, L), jnp.float32),
            pltpu.VMEM((_G_CH, L), jnp.float32),
            pltpu.VMEM((_G_CH, L), jnp.float32),
            pltpu.VMEM((_G_CH, L), jnp.float32),
            pltpu.SemaphoreType.DMA,
            pltpu.SemaphoreType.DMA,
            pltpu.SemaphoreType.DMA,
            pltpu.SemaphoreType.DMA,
            pltpu.SemaphoreType.DMA,
        ],
    )
    def k(table_hbm, idx1_hbm, idx2_hbm, out_hbm, i1_v, i2_v,
          r1a, r2a, r1b, r2b, s1a, s2a, s1b, s2b, so):
        wid = lax.axis_index("s") * NC + lax.axis_index("c")
        base = wid * per_w
        pltpu.sync_copy(idx1_hbm.at[pl.ds(base, per_w)], i1_v)
        pltpu.sync_copy(idx2_hbm.at[pl.ds(base, per_w)], i2_v)

        def addloop(ra, rb):
            def arow(j, _):
                for c in range(L // 16):
                    sl = pl.ds(c * 16, 16)
                    ra[j, sl] = ra[j, sl] + rb[j, sl]
                return 0
            lax.fori_loop(0, _G_CH, arow, 0, unroll=False)

        def outer(j, _):
            o0 = j * 2 * _G_CH
            d1a = pltpu.async_copy(
                table_hbm.at[i1_v.at[pl.ds(o0, _G_CH)]], r1a, s1a)
            d2a = pltpu.async_copy(
                table_hbm.at[i2_v.at[pl.ds(o0, _G_CH)]], r2a, s2a)
            d1b = pltpu.async_copy(
                table_hbm.at[i1_v.at[pl.ds(o0 + _G_CH, _G_CH)]], r1b, s1b)
            d2b = pltpu.async_copy(
                table_hbm.at[i2_v.at[pl.ds(o0 + _G_CH, _G_CH)]], r2b, s2b)
            d1a.wait()
            d2a.wait()
            addloop(r1a, r2a)
            wa = pltpu.async_copy(r1a, out_hbm.at[pl.ds(base + o0, _G_CH)], so)
            d1b.wait()
            d2b.wait()
            addloop(r1b, r2b)
            wb = pltpu.async_copy(
                r1b, out_hbm.at[pl.ds(base + o0 + _G_CH, _G_CH)], so)
            wa.wait()
            wb.wait()
            return 0

        lax.fori_loop(0, nch // 2, outer, 0, unroll=False)

    return k(table, idx1, idx2)


_S_CH = 200
NP = 10240
_ZROWS = 128


def _sc_scatter(vals, dst):
    r = vals.shape[0]
    per_c = r // NC
    per_w = r // NW
    nch = per_w // _S_CH

    @functools.partial(
        pl.kernel,
        out_type=jax.ShapeDtypeStruct((NC, NP, L), jnp.float32),
        mesh=_sc_mesh(),
        scratch_types=[
            pltpu.VMEM((_S_CH,), jnp.int32),
            pltpu.VMEM((_S_CH, L), jnp.float32),
            pltpu.VMEM((_ZROWS, L), jnp.float32),
            pltpu.VMEM_SHARED((NP, L), jnp.float32),
        ],
    )
    def k(vals_hbm, dst_hbm, out_hbm, idx_v, rows_v, zbuf, acc):
        c = lax.axis_index("c")
        s = lax.axis_index("s")
        z16 = jnp.zeros((16,), jnp.float32)

        def zrow(i, _):
            for j in range(L // 16):
                zbuf[i, pl.ds(j * 16, 16)] = z16
            return 0

        lax.fori_loop(0, _ZROWS, zrow, 0, unroll=False)
        for j in range(NP // (NS * _ZROWS)):
            pltpu.sync_copy(
                zbuf, acc.at[pl.ds(s * (NP // NS) + j * _ZROWS, _ZROWS)])
        plsc.subcore_barrier()

        def body(i, _):
            b = c * per_c + s * per_w + i * _S_CH
            pltpu.sync_copy(dst_hbm.at[pl.ds(b, _S_CH)], idx_v)
            pltpu.sync_copy(vals_hbm.at[pl.ds(b, _S_CH)], rows_v)
            pltpu.sync_copy(rows_v, acc.at[idx_v], add=True)
            return 0

        lax.fori_loop(0, nch, body, 0, unroll=False)
        plsc.subcore_barrier()
        pltpu.sync_copy(acc.at[pl.ds(s * (NP // NS), NP // NS)],
                        out_hbm.at[c, pl.ds(s * (NP // NS), NP // NS)])

    return k(vals, dst)


def kernel(node_feat, edge_index, mesh_edge_attr, world_edge_index,
           world_edge_attr, enw1, enb1, enw2, enb2, eng, enbb, emw1, emb1,
           emw2, emb2, emg, embb, eww1, ewb1, eww2, ewb2, ewg, ewbb, pmw1,
           pmb1, pmw2, pmb2, pmg, pmbb, pww1, pwb1, pww2, pwb2, pwg, pwbb,
           pnw1, pnb1, pnw2, pnb2, png, pnbb, dw1, db1, dw2, db2):
    se, de = edge_index[0], edge_index[1]
    ws, wr = world_edge_index[0], world_edge_index[1]
    idx_src = jnp.concatenate([se, ws + 2 * N]).astype(jnp.int32)
    idx_dst = jnp.concatenate([de + N, wr + 3 * N]).astype(jnp.int32)
    de32 = de.astype(jnp.int32)
    wr32 = wr.astype(jnp.int32)

    row2 = lambda a: a.reshape(1, L)

    nf = jnp.pad(node_feat, ((0, 0), (0, 16 - node_feat.shape[1])))
    enw1p = jnp.pad(enw1, ((0, 16 - enw1.shape[0]), (0, 0)))
    node = _enc_mlp(nf, enw1p, row2(enb1), enw2, row2(enb2), row2(eng),
                    row2(enbb), 2000)
    mea = jnp.pad(mesh_edge_attr, ((0, 0), (0, 16 - mesh_edge_attr.shape[1])))
    emw1p = jnp.pad(emw1, ((0, 16 - emw1.shape[0]), (0, 0)))
    me = _enc_mlp(mea, emw1p, row2(emb1), emw2, row2(emb2), row2(emg),
                  row2(embb), 4000)
    wea = jnp.pad(world_edge_attr, ((0, 0), (0, 16 - world_edge_attr.shape[1])))
    eww1p = jnp.pad(eww1, ((0, 16 - eww1.shape[0]), (0, 0)))
    we = _enc_mlp(wea, eww1p, row2(ewb1), eww2, row2(ewb2), row2(ewg),
                  row2(ewbb), 4000)

    xs = dict(
        wproj=jnp.stack([pmw1[:, L:2 * L], pmw1[:, 2 * L:], pww1[:, L:2 * L],
                         pww1[:, 2 * L:]], axis=1),
        am=pmw1[:, :L], aw=pww1[:, :L],
        pmb1=pmb1[:, None, :], pmb2=pmb2[:, None, :], pmw2=pmw2,
        pmg=pmg[:, None, :], pmbb=pmbb[:, None, :],
        pwb1=pwb1[:, None, :], pwb2=pwb2[:, None, :], pww2=pww2,
        pwg=pwg[:, None, :], pwbb=pwbb[:, None, :],
        pa=pnw1[:, :L], pm=pnw1[:, L:2 * L], pw=pnw1[:, 2 * L:],
        pnb1=pnb1[:, None, :], pnb2=pnb2[:, None, :], pnw2=pnw2,
        png=png[:, None, :], pnbb=pnbb[:, None, :],
    )

    def step(carry, w):
        node, me, we = carry
        tbl = _proj(node, w['wproj'], 2000)
        gsum = _sc_gather_add(tbl, idx_src, idx_dst)
        tme = _mm_bias(me, w['am'], w['pmb1'], 2000)
        twe = _mm_bias(we, w['aw'], w['pwb1'], 2000)
        tnd = _mm_bias(node, w['pa'], w['pnb1'], 2000)
        me = _edge_mlp(me, tme, gsum, 0, w['pmw2'],
                       w['pmb2'], w['pmg'], w['pmbb'], 2000)
        we = _edge_mlp(we, twe, gsum, E, w['pww2'],
                       w['pwb2'], w['pwg'], w['pwbb'], 2000)
        magg = _sc_scatter(me, de32)
        wagg = _sc_scatter(we, wr32)
        node = _node_mlp(node, tnd, magg, wagg, w['pm'], w['pw'],
                         w['pnw2'], w['pnb2'], w['png'],
                         w['pnbb'], 2000)
        return (node, me, we), None

    (node, me, we), _ = lax.scan(step, (node, me, we), xs)

    dw2p = jnp.pad(dw2, ((0, 0), (0, L - dw2.shape[1])))
    db2p = jnp.pad(db2, ((0, L - db2.shape[0]),)).reshape(1, L)
    out = _decoder(node, dw1, row2(db1), dw2p, db2p, 2000)
    return out[:, :dw2.shape[1]]


# speedup vs baseline: 1.1419x; 1.1419x over previous
import functools

import jax
import jax.numpy as jnp
from jax import lax
from jax.experimental import pallas as pl
from jax.experimental.pallas import tpu as pltpu
from jax.experimental.pallas import tpu_sc as plsc

N = 10000
E = 160000
WE = 32000
L = 128
S = 15

NC = 2
NS = 16
NW = NC * NS


def _ln_rows(y, g, bb):
    m = jnp.mean(y, axis=-1, keepdims=True)
    v = jnp.mean((y - m) ** 2, axis=-1, keepdims=True)
    return (y - m) * lax.rsqrt(v + 1e-5) * g + bb


def _enc_body(x_ref, w1_ref, b1_ref, w2_ref, b2_ref, g_ref, bb_ref, o_ref):
    x = x_ref[...]
    h = jnp.maximum(
        jnp.dot(x, w1_ref[...], preferred_element_type=jnp.float32) + b1_ref[...], 0.0)
    y = jnp.dot(h, w2_ref[...], preferred_element_type=jnp.float32) + b2_ref[...]
    o_ref[...] = _ln_rows(y, g_ref[...], bb_ref[...])


def _enc_mlp(x, w1, b1, w2, b2, g, bb, br):
    r, k = x.shape
    full = lambda *s: pl.BlockSpec(s, lambda i: tuple(0 for _ in s))
    return pl.pallas_call(
        _enc_body,
        grid=(r // br,),
        in_specs=[
            pl.BlockSpec((br, k), lambda i: (i, 0)),
            full(k, L), full(1, L), full(L, L), full(1, L), full(1, L), full(1, L),
        ],
        out_specs=pl.BlockSpec((br, L), lambda i: (i, 0)),
        out_shape=jax.ShapeDtypeStruct((r, L), jnp.float32),
    )(x, w1, b1, w2, b2, g, bb)


def _pack_bf16_pair(lo, hi):
    ulo = lax.bitcast_convert_type(lo, jnp.uint32)
    uhi = lax.bitcast_convert_type(hi, jnp.uint32)
    rlo = (ulo + jnp.uint32(0x7FFF) + ((ulo >> 16) & jnp.uint32(1))) >> 16
    rhi = ((uhi + jnp.uint32(0x7FFF) + ((uhi >> 16) & jnp.uint32(1)))
           & jnp.uint32(0xFFFF0000))
    return rlo | rhi


def _unpack_bf16_pair(u):
    lo = lax.bitcast_convert_type(u << 16, jnp.float32)
    hi = lax.bitcast_convert_type(u & jnp.uint32(0xFFFF0000), jnp.float32)
    return jnp.concatenate([lo, hi], axis=-1)


def _proj_body(x_ref, w_ref, o_ref):
    o_ref[...] = jnp.dot(x_ref[...], w_ref[0],
                         preferred_element_type=jnp.float32)


def _proj(node, wstack, bn):
    nb = N // bn
    return pl.pallas_call(
        _proj_body,
        grid=(4, nb),
        in_specs=[
            pl.BlockSpec((bn, L), lambda t, i: (i, 0)),
            pl.BlockSpec((1, L, L), lambda t, i: (t, 0, 0)),
        ],
        out_specs=pl.BlockSpec((bn, L), lambda t, i: (t * nb + i, 0)),
        out_shape=jax.ShapeDtypeStruct((4 * N, L), jnp.float32),
    )(node, wstack)


def _edge_body(x_ref, g12_ref, a_ref, b1_ref, w2_ref, b2_ref,
               g_ref, bb_ref, o_ref):
    x = x_ref[...]
    t = jnp.dot(x, a_ref[...], preferred_element_type=jnp.float32)
    h = jnp.maximum(t + g12_ref[...] + b1_ref[...], 0.0)
    y = jnp.dot(h, w2_ref[...],
                preferred_element_type=jnp.float32) + b2_ref[...]
    o_ref[...] = x + _ln_rows(y, g_ref[...], bb_ref[...])


def _edge_mlp(x, gsum, off, a, b1, w2, b2, g, bb, be):
    r = x.shape[0]
    full = lambda *s: pl.BlockSpec(s, lambda i: tuple(0 for _ in s))
    o1 = off // be
    return pl.pallas_call(
        _edge_body,
        grid=(r // be,),
        in_specs=[
            pl.BlockSpec((be, L), lambda i: (i, 0)),
            pl.BlockSpec((be, L), lambda i, o=o1: (o + i, 0)),
            full(L, L), full(1, L), full(L, L), full(1, L), full(1, L), full(1, L),
        ],
        out_specs=pl.BlockSpec((be, L), lambda i: (i, 0)),
        out_shape=jax.ShapeDtypeStruct((r, L), jnp.float32),
    )(x, gsum, a, b1, w2, b2, g, bb)


def _node_body(x_ref, m0_ref, m1_ref, w0_ref, w1_ref, pa_ref, pm_ref, pw_ref,
               b1_ref, w2_ref, b2_ref, g_ref, bb_ref, o_ref):
    x = x_ref[...]
    mg = m0_ref[0] + m1_ref[0]
    wg = w0_ref[0] + w1_ref[0]
    t = (jnp.dot(x, pa_ref[...], preferred_element_type=jnp.float32)
         + jnp.dot(mg, pm_ref[...], preferred_element_type=jnp.float32)
         + jnp.dot(wg, pw_ref[...], preferred_element_type=jnp.float32)
         + b1_ref[...])
    h = jnp.maximum(t, 0.0)
    y = jnp.dot(h, w2_ref[...],
                preferred_element_type=jnp.float32) + b2_ref[...]
    o_ref[...] = x + _ln_rows(y, g_ref[...], bb_ref[...])


def _node_mlp(node, magg, wagg, pa, pm, pw, b1, w2, b2, g, bb, bn):
    full = lambda *s: pl.BlockSpec(s, lambda i: tuple(0 for _ in s))
    part = lambda c: pl.BlockSpec((1, bn, L), lambda i, c=c: (c, i, 0))
    return pl.pallas_call(
        _node_body,
        grid=(N // bn,),
        in_specs=[
            pl.BlockSpec((bn, L), lambda i: (i, 0)),
            part(0), part(1), part(0), part(1),
            full(L, L), full(L, L), full(L, L),
            full(1, L), full(L, L), full(1, L), full(1, L), full(1, L),
        ],
        out_specs=pl.BlockSpec((bn, L), lambda i: (i, 0)),
        out_shape=jax.ShapeDtypeStruct((N, L), jnp.float32),
    )(node, magg, magg, wagg, wagg, pa, pm, pw, b1, w2, b2, g, bb)


def _dec_body(x_ref, w1_ref, b1_ref, w2_ref, b2_ref, o_ref):
    h = jnp.maximum(
        jnp.dot(x_ref[...], w1_ref[...], preferred_element_type=jnp.float32)
        + b1_ref[...], 0.0)
    o_ref[...] = jnp.dot(h, w2_ref[...],
                         preferred_element_type=jnp.float32) + b2_ref[...]


def _decoder(node, w1, b1, w2p, b2p, bn):
    full = lambda *s: pl.BlockSpec(s, lambda i: tuple(0 for _ in s))
    return pl.pallas_call(
        _dec_body,
        grid=(N // bn,),
        in_specs=[pl.BlockSpec((bn, L), lambda i: (i, 0)),
                  full(L, L), full(1, L), full(L, L), full(1, L)],
        out_specs=pl.BlockSpec((bn, L), lambda i: (i, 0)),
        out_shape=jax.ShapeDtypeStruct((N, L), jnp.float32),
    )(node, w1, b1, w2p, b2p)


def _sc_mesh():
    return plsc.VectorSubcoreMesh(core_axis_name="c", subcore_axis_name="s",
                                  num_cores=NC, num_subcores=NS)

_G_CH = 200


def _sc_gather_add(table, idx1, idx2):
    r = idx1.shape[0]
    per_w = r // NW
    nch = per_w // _G_CH
    assert nch % 2 == 0

    @functools.partial(
        pl.kernel,
        out_type=jax.ShapeDtypeStruct((r, L), jnp.float32),
        mesh=_sc_mesh(),
        scratch_types=[
            pltpu.VMEM((per_w,), jnp.int32),
            pltpu.VMEM((per_w,), jnp.int32),
            pltpu.VMEM((_G_CH, L), jnp.float32),
            pltpu.VMEM((_G_CH, L), jnp.float32),
            pltpu.VMEM((_G_CH, L), jnp.float32),
            pltpu.VMEM((_G_CH, L), jnp.float32),
            pltpu.SemaphoreType.DMA,
            pltpu.SemaphoreType.DMA,
            pltpu.SemaphoreType.DMA,
            pltpu.SemaphoreType.DMA,
            pltpu.SemaphoreType.DMA,
        ],
    )
    def k(table_hbm, idx1_hbm, idx2_hbm, out_hbm, i1_v, i2_v,
          r1a, r2a, r1b, r2b, s1a, s2a, s1b, s2b, so):
        wid = lax.axis_index("s") * NC + lax.axis_index("c")
        base = wid * per_w
        pltpu.sync_copy(idx1_hbm.at[pl.ds(base, per_w)], i1_v)
        pltpu.sync_copy(idx2_hbm.at[pl.ds(base, per_w)], i2_v)

        def addloop(ra, rb):
            def arow(j, _):
                for c in range(L // 16):
                    sl = pl.ds(c * 16, 16)
                    ra[j, sl] = ra[j, sl] + rb[j, sl]
                return 0
            lax.fori_loop(0, _G_CH, arow, 0, unroll=False)

        def outer(j, _):
            o0 = j * 2 * _G_CH
            d1a = pltpu.async_copy(
                table_hbm.at[i1_v.at[pl.ds(o0, _G_CH)]], r1a, s1a)
            d2a = pltpu.async_copy(
                table_hbm.at[i2_v.at[pl.ds(o0, _G_CH)]], r2a, s2a)
            d1b = pltpu.async_copy(
                table_hbm.at[i1_v.at[pl.ds(o0 + _G_CH, _G_CH)]], r1b, s1b)
            d2b = pltpu.async_copy(
                table_hbm.at[i2_v.at[pl.ds(o0 + _G_CH, _G_CH)]], r2b, s2b)
            d1a.wait()
            d2a.wait()
            addloop(r1a, r2a)
            wa = pltpu.async_copy(r1a, out_hbm.at[pl.ds(base + o0, _G_CH)], so)
            d1b.wait()
            d2b.wait()
            addloop(r1b, r2b)
            wb = pltpu.async_copy(
                r1b, out_hbm.at[pl.ds(base + o0 + _G_CH, _G_CH)], so)
            wa.wait()
            wb.wait()
            return 0

        lax.fori_loop(0, nch // 2, outer, 0, unroll=False)

    return k(table, idx1, idx2)


_S_CH = 200
NP = 10240
_ZROWS = 128


def _sc_scatter(vals, dst):
    r = vals.shape[0]
    per_c = r // NC
    per_w = r // NW
    nch = per_w // _S_CH

    @functools.partial(
        pl.kernel,
        out_type=jax.ShapeDtypeStruct((NC, NP, L), jnp.float32),
        mesh=_sc_mesh(),
        scratch_types=[
            pltpu.VMEM((_S_CH,), jnp.int32),
            pltpu.VMEM((_S_CH, L), jnp.float32),
            pltpu.VMEM((_ZROWS, L), jnp.float32),
            pltpu.VMEM_SHARED((NP, L), jnp.float32),
        ],
    )
    def k(vals_hbm, dst_hbm, out_hbm, idx_v, rows_v, zbuf, acc):
        c = lax.axis_index("c")
        s = lax.axis_index("s")
        z16 = jnp.zeros((16,), jnp.float32)

        def zrow(i, _):
            for j in range(L // 16):
                zbuf[i, pl.ds(j * 16, 16)] = z16
            return 0

        lax.fori_loop(0, _ZROWS, zrow, 0, unroll=False)
        for j in range(NP // (NS * _ZROWS)):
            pltpu.sync_copy(
                zbuf, acc.at[pl.ds(s * (NP // NS) + j * _ZROWS, _ZROWS)])
        plsc.subcore_barrier()

        def body(i, _):
            b = c * per_c + s * per_w + i * _S_CH
            pltpu.sync_copy(dst_hbm.at[pl.ds(b, _S_CH)], idx_v)
            pltpu.sync_copy(vals_hbm.at[pl.ds(b, _S_CH)], rows_v)
            pltpu.sync_copy(rows_v, acc.at[idx_v], add=True)
            return 0

        lax.fori_loop(0, nch, body, 0, unroll=False)
        plsc.subcore_barrier()
        pltpu.sync_copy(acc.at[pl.ds(s * (NP // NS), NP // NS)],
                        out_hbm.at[c, pl.ds(s * (NP // NS), NP // NS)])

    return k(vals, dst)


def kernel(node_feat, edge_index, mesh_edge_attr, world_edge_index,
           world_edge_attr, enw1, enb1, enw2, enb2, eng, enbb, emw1, emb1,
           emw2, emb2, emg, embb, eww1, ewb1, eww2, ewb2, ewg, ewbb, pmw1,
           pmb1, pmw2, pmb2, pmg, pmbb, pww1, pwb1, pww2, pwb2, pwg, pwbb,
           pnw1, pnb1, pnw2, pnb2, png, pnbb, dw1, db1, dw2, db2):
    se, de = edge_index[0], edge_index[1]
    ws, wr = world_edge_index[0], world_edge_index[1]
    idx_src = jnp.concatenate([se, ws + 2 * N]).astype(jnp.int32)
    idx_dst = jnp.concatenate([de + N, wr + 3 * N]).astype(jnp.int32)
    de32 = de.astype(jnp.int32)
    wr32 = wr.astype(jnp.int32)

    row2 = lambda a: a.reshape(1, L)

    nf = jnp.pad(node_feat, ((0, 0), (0, 16 - node_feat.shape[1])))
    enw1p = jnp.pad(enw1, ((0, 16 - enw1.shape[0]), (0, 0)))
    node = _enc_mlp(nf, enw1p, row2(enb1), enw2, row2(enb2), row2(eng),
                    row2(enbb), 2000)
    mea = jnp.pad(mesh_edge_attr, ((0, 0), (0, 16 - mesh_edge_attr.shape[1])))
    emw1p = jnp.pad(emw1, ((0, 16 - emw1.shape[0]), (0, 0)))
    me = _enc_mlp(mea, emw1p, row2(emb1), emw2, row2(emb2), row2(emg),
                  row2(embb), 4000)
    wea = jnp.pad(world_edge_attr, ((0, 0), (0, 16 - world_edge_attr.shape[1])))
    eww1p = jnp.pad(eww1, ((0, 16 - eww1.shape[0]), (0, 0)))
    we = _enc_mlp(wea, eww1p, row2(ewb1), eww2, row2(ewb2), row2(ewg),
                  row2(ewbb), 4000)

    xs = dict(
        wproj=jnp.stack([pmw1[:, L:2 * L], pmw1[:, 2 * L:], pww1[:, L:2 * L],
                         pww1[:, 2 * L:]], axis=1),
        am=pmw1[:, :L], aw=pww1[:, :L],
        pmb1=pmb1[:, None, :], pmb2=pmb2[:, None, :], pmw2=pmw2,
        pmg=pmg[:, None, :], pmbb=pmbb[:, None, :],
        pwb1=pwb1[:, None, :], pwb2=pwb2[:, None, :], pww2=pww2,
        pwg=pwg[:, None, :], pwbb=pwbb[:, None, :],
        pa=pnw1[:, :L], pm=pnw1[:, L:2 * L], pw=pnw1[:, 2 * L:],
        pnb1=pnb1[:, None, :], pnb2=pnb2[:, None, :], pnw2=pnw2,
        png=png[:, None, :], pnbb=pnbb[:, None, :],
    )

    def step(carry, w):
        node, me, we = carry
        tbl = _proj(node, w['wproj'], 2000)
        gsum = _sc_gather_add(tbl, idx_src, idx_dst)
        me = _edge_mlp(me, gsum, 0, w['am'], w['pmb1'], w['pmw2'],
                       w['pmb2'], w['pmg'], w['pmbb'], 4000)
        we = _edge_mlp(we, gsum, E, w['aw'], w['pwb1'],
                       w['pww2'], w['pwb2'], w['pwg'], w['pwbb'], 4000)
        magg = _sc_scatter(me, de32)
        wagg = _sc_scatter(we, wr32)
        node = _node_mlp(node, magg, wagg, w['pa'], w['pm'], w['pw'],
                         w['pnb1'], w['pnw2'], w['pnb2'], w['png'],
                         w['pnbb'], 2000)
        return (node, me, we), None

    (node, me, we), _ = lax.scan(step, (node, me, we), xs)

    dw2p = jnp.pad(dw2, ((0, 0), (0, L - dw2.shape[1])))
    db2p = jnp.pad(db2, ((0, L - db2.shape[0]),)).reshape(1, L)
    out = _decoder(node, dw1, row2(db1), dw2p, db2p, 2000)
    return out[:, :dw2.shape[1]]


# speedup vs baseline: 1.1799x; 1.0333x over previous
import functools

import jax
import jax.numpy as jnp
from jax import lax
from jax.experimental import pallas as pl
from jax.experimental.pallas import tpu as pltpu
from jax.experimental.pallas import tpu_sc as plsc

N = 10000
E = 160000
WE = 32000
L = 128
S = 15

NC = 2
NS = 16
NW = NC * NS


def _ln_rows(y, g, bb):
    m = jnp.mean(y, axis=-1, keepdims=True)
    v = jnp.mean((y - m) ** 2, axis=-1, keepdims=True)
    return (y - m) * lax.rsqrt(v + 1e-5) * g + bb


def _enc_body(x_ref, w1_ref, b1_ref, w2_ref, b2_ref, g_ref, bb_ref, o_ref):
    x = x_ref[...]
    h = jnp.maximum(
        jnp.dot(x, w1_ref[...], preferred_element_type=jnp.float32) + b1_ref[...], 0.0)
    y = jnp.dot(h, w2_ref[...], preferred_element_type=jnp.float32) + b2_ref[...]
    o_ref[...] = _ln_rows(y, g_ref[...], bb_ref[...])


def _enc_mlp(x, w1, b1, w2, b2, g, bb, br):
    r, k = x.shape
    full = lambda *s: pl.BlockSpec(s, lambda i: tuple(0 for _ in s))
    return pl.pallas_call(
        _enc_body,
        grid=(r // br,),
        in_specs=[
            pl.BlockSpec((br, k), lambda i: (i, 0)),
            full(k, L), full(1, L), full(L, L), full(1, L), full(1, L), full(1, L),
        ],
        out_specs=pl.BlockSpec((br, L), lambda i: (i, 0)),
        out_shape=jax.ShapeDtypeStruct((r, L), jnp.float32),
    )(x, w1, b1, w2, b2, g, bb)


def _pack_bf16_pair(lo, hi):
    ulo = lax.bitcast_convert_type(lo, jnp.uint32)
    uhi = lax.bitcast_convert_type(hi, jnp.uint32)
    rlo = (ulo + jnp.uint32(0x7FFF) + ((ulo >> 16) & jnp.uint32(1))) >> 16
    rhi = ((uhi + jnp.uint32(0x7FFF) + ((uhi >> 16) & jnp.uint32(1)))
           & jnp.uint32(0xFFFF0000))
    return rlo | rhi


def _unpack_bf16_pair(u):
    lo = lax.bitcast_convert_type(u << 16, jnp.float32)
    hi = lax.bitcast_convert_type(u & jnp.uint32(0xFFFF0000), jnp.float32)
    return jnp.concatenate([lo, hi], axis=-1)


def _proj_body(x_ref, w_ref, o_ref):
    o_ref[...] = jnp.dot(x_ref[...], w_ref[0],
                         preferred_element_type=jnp.float32)


def _proj(node, wstack, bn):
    nb = N // bn
    return pl.pallas_call(
        _proj_body,
        grid=(4, nb),
        in_specs=[
            pl.BlockSpec((bn, L), lambda t, i: (i, 0)),
            pl.BlockSpec((1, L, L), lambda t, i: (t, 0, 0)),
        ],
        out_specs=pl.BlockSpec((bn, L), lambda t, i: (t * nb + i, 0)),
        out_shape=jax.ShapeDtypeStruct((4 * N, L), jnp.float32),
    )(node, wstack)


def _edge_body(x_ref, g12_ref, a_ref, b1_ref, w2_ref, b2_ref,
               g_ref, bb_ref, o_ref):
    x = x_ref[...]
    t = jnp.dot(x, a_ref[...], preferred_element_type=jnp.float32)
    h = jnp.maximum(t + g12_ref[...] + b1_ref[...], 0.0)
    y = jnp.dot(h, w2_ref[...],
                preferred_element_type=jnp.float32) + b2_ref[...]
    o_ref[...] = x + _ln_rows(y, g_ref[...], bb_ref[...])


def _edge_mlp(x, gsum, off, a, b1, w2, b2, g, bb, be):
    r = x.shape[0]
    full = lambda *s: pl.BlockSpec(s, lambda i: tuple(0 for _ in s))
    o1 = off // be
    return pl.pallas_call(
        _edge_body,
        grid=(r // be,),
        in_specs=[
            pl.BlockSpec((be, L), lambda i: (i, 0)),
            pl.BlockSpec((be, L), lambda i, o=o1: (o + i, 0)),
            full(L, L), full(1, L), full(L, L), full(1, L), full(1, L), full(1, L),
        ],
        out_specs=pl.BlockSpec((be, L), lambda i: (i, 0)),
        out_shape=jax.ShapeDtypeStruct((r, L), jnp.float32),
    )(x, gsum, a, b1, w2, b2, g, bb)


def _node_body(x_ref, m0_ref, m1_ref, w0_ref, w1_ref, pa_ref, pm_ref, pw_ref,
               b1_ref, w2_ref, b2_ref, g_ref, bb_ref, o_ref):
    x = x_ref[...]
    mg = m0_ref[0] + m1_ref[0]
    wg = w0_ref[0] + w1_ref[0]
    t = (jnp.dot(x, pa_ref[...], preferred_element_type=jnp.float32)
         + jnp.dot(mg, pm_ref[...], preferred_element_type=jnp.float32)
         + jnp.dot(wg, pw_ref[...], preferred_element_type=jnp.float32)
         + b1_ref[...])
    h = jnp.maximum(t, 0.0)
    y = jnp.dot(h, w2_ref[...],
                preferred_element_type=jnp.float32) + b2_ref[...]
    o_ref[...] = x + _ln_rows(y, g_ref[...], bb_ref[...])


def _node_mlp(node, magg, wagg, pa, pm, pw, b1, w2, b2, g, bb, bn):
    full = lambda *s: pl.BlockSpec(s, lambda i: tuple(0 for _ in s))
    part = lambda c: pl.BlockSpec((1, bn, L), lambda i, c=c: (c, i, 0))
    return pl.pallas_call(
        _node_body,
        grid=(N // bn,),
        in_specs=[
            pl.BlockSpec((bn, L), lambda i: (i, 0)),
            part(0), part(1), part(0), part(1),
            full(L, L), full(L, L), full(L, L),
            full(1, L), full(L, L), full(1, L), full(1, L), full(1, L),
        ],
        out_specs=pl.BlockSpec((bn, L), lambda i: (i, 0)),
        out_shape=jax.ShapeDtypeStruct((N, L), jnp.float32),
    )(node, magg, magg, wagg, wagg, pa, pm, pw, b1, w2, b2, g, bb)


def _dec_body(x_ref, w1_ref, b1_ref, w2_ref, b2_ref, o_ref):
    h = jnp.maximum(
        jnp.dot(x_ref[...], w1_ref[...], preferred_element_type=jnp.float32)
        + b1_ref[...], 0.0)
    o_ref[...] = jnp.dot(h, w2_ref[...],
                         preferred_element_type=jnp.float32) + b2_ref[...]


def _decoder(node, w1, b1, w2p, b2p, bn):
    full = lambda *s: pl.BlockSpec(s, lambda i: tuple(0 for _ in s))
    return pl.pallas_call(
        _dec_body,
        grid=(N // bn,),
        in_specs=[pl.BlockSpec((bn, L), lambda i: (i, 0)),
                  full(L, L), full(1, L), full(L, L), full(1, L)],
        out_specs=pl.BlockSpec((bn, L), lambda i: (i, 0)),
        out_shape=jax.ShapeDtypeStruct((N, L), jnp.float32),
    )(node, w1, b1, w2p, b2p)


def _sc_mesh():
    return plsc.VectorSubcoreMesh(core_axis_name="c", subcore_axis_name="s",
                                  num_cores=NC, num_subcores=NS)

_G_CH = 200


def _sc_gather_add(table, idx1, idx2):
    r = idx1.shape[0]
    per_w = r // NW
    nch = per_w // _G_CH
    assert nch % 2 == 0

    @functools.partial(
        pl.kernel,
        out_type=jax.ShapeDtypeStruct((r, L), jnp.float32),
        mesh=_sc_mesh(),
        scratch_types=[
            pltpu.VMEM((per_w,), jnp.int32),
            pltpu.VMEM((per_w,), jnp.int32),
            pltpu.VMEM((_G_CH, L), jnp.float32),
            pltpu.VMEM((_G_CH, L), jnp.float32),
            pltpu.VMEM((_G_CH, L), jnp.float32),
            pltpu.VMEM((_G_CH, L), jnp.float32),
            pltpu.SemaphoreType.DMA,
            pltpu.SemaphoreType.DMA,
            pltpu.SemaphoreType.DMA,
            pltpu.SemaphoreType.DMA,
            pltpu.SemaphoreType.DMA,
        ],
    )
    def k(table_hbm, idx1_hbm, idx2_hbm, out_hbm, i1_v, i2_v,
          r1a, r2a, r1b, r2b, s1a, s2a, s1b, s2b, so):
        wid = lax.axis_index("s") * NC + lax.axis_index("c")
        base = wid * per_w
        pltpu.sync_copy(idx1_hbm.at[pl.ds(base, per_w)], i1_v)
        pltpu.sync_copy(idx2_hbm.at[pl.ds(base, per_w)], i2_v)

        def addloop(ra, rb):
            def arow(j, _):
                for c in range(L // 16):
                    sl = pl.ds(c * 16, 16)
                    ra[j, sl] = ra[j, sl] + rb[j, sl]
                return 0
            lax.fori_loop(0, _G_CH, arow, 0, unroll=False)

        def outer(j, _):
            o0 = j * 2 * _G_CH
            d1a = pltpu.async_copy(
                table_hbm.at[i1_v.at[pl.ds(o0, _G_CH)]], r1a, s1a)
            d2a = pltpu.async_copy(
                table_hbm.at[i2_v.at[pl.ds(o0, _G_CH)]], r2a, s2a)
            d1b = pltpu.async_copy(
                table_hbm.at[i1_v.at[pl.ds(o0 + _G_CH, _G_CH)]], r1b, s1b)
            d2b = pltpu.async_copy(
                table_hbm.at[i2_v.at[pl.ds(o0 + _G_CH, _G_CH)]], r2b, s2b)
            d1a.wait()
            d2a.wait()
            addloop(r1a, r2a)
            wa = pltpu.async_copy(r1a, out_hbm.at[pl.ds(base + o0, _G_CH)], so)
            d1b.wait()
            d2b.wait()
            addloop(r1b, r2b)
            wb = pltpu.async_copy(
                r1b, out_hbm.at[pl.ds(base + o0 + _G_CH, _G_CH)], so)
            wa.wait()
            wb.wait()
            return 0

        lax.fori_loop(0, nch // 2, outer, 0, unroll=False)

    return k(table, idx1, idx2)


_S_CH = 200
NP = 10240
_ZROWS = 128


def _sc_scatter(vals, dst):
    r = vals.shape[0]
    per_c = r // NC
    per_w = r // NW
    nch = per_w // _S_CH

    @functools.partial(
        pl.kernel,
        out_type=jax.ShapeDtypeStruct((NC, NP, L), jnp.float32),
        mesh=_sc_mesh(),
        scratch_types=[
            pltpu.VMEM((_S_CH,), jnp.int32),
            pltpu.VMEM((_S_CH, L), jnp.float32),
            pltpu.VMEM((_ZROWS, L), jnp.float32),
            pltpu.VMEM_SHARED((NP, L), jnp.float32),
        ],
    )
    def k(vals_hbm, dst_hbm, out_hbm, idx_v, rows_v, zbuf, acc):
        c = lax.axis_index("c")
        s = lax.axis_index("s")
        z16 = jnp.zeros((16,), jnp.float32)

        def zrow(i, _):
            for j in range(L // 16):
                zbuf[i, pl.ds(j * 16, 16)] = z16
            return 0

        lax.fori_loop(0, _ZROWS, zrow, 0, unroll=False)
        for j in range(NP // (NS * _ZROWS)):
            pltpu.sync_copy(
                zbuf, acc.at[pl.ds(s * (NP // NS) + j * _ZROWS, _ZROWS)])
        plsc.subcore_barrier()

        def body(i, _):
            b = c * per_c + s * per_w + i * _S_CH
            pltpu.sync_copy(dst_hbm.at[pl.ds(b, _S_CH)], idx_v)
            pltpu.sync_copy(vals_hbm.at[pl.ds(b, _S_CH)], rows_v)
            pltpu.sync_copy(rows_v, acc.at[idx_v], add=True)
            return 0

        lax.fori_loop(0, nch, body, 0, unroll=False)
        plsc.subcore_barrier()
        pltpu.sync_copy(acc.at[pl.ds(s * (NP // NS), NP // NS)],
                        out_hbm.at[c, pl.ds(s * (NP // NS), NP // NS)])

    return k(vals, dst)


def kernel(node_feat, edge_index, mesh_edge_attr, world_edge_index,
           world_edge_attr, enw1, enb1, enw2, enb2, eng, enbb, emw1, emb1,
           emw2, emb2, emg, embb, eww1, ewb1, eww2, ewb2, ewg, ewbb, pmw1,
           pmb1, pmw2, pmb2, pmg, pmbb, pww1, pwb1, pww2, pwb2, pwg, pwbb,
           pnw1, pnb1, pnw2, pnb2, png, pnbb, dw1, db1, dw2, db2):
    se, de = edge_index[0], edge_index[1]
    ws, wr = world_edge_index[0], world_edge_index[1]
    idx_src = jnp.concatenate([se, ws + 2 * N]).astype(jnp.int32)
    idx_dst = jnp.concatenate([de + N, wr + 3 * N]).astype(jnp.int32)
    de32 = de.astype(jnp.int32)
    wr32 = wr.astype(jnp.int32)

    row2 = lambda a: a.reshape(1, L)

    nf = jnp.pad(node_feat, ((0, 0), (0, 16 - node_feat.shape[1])))
    enw1p = jnp.pad(enw1, ((0, 16 - enw1.shape[0]), (0, 0)))
    node = _enc_mlp(nf, enw1p, row2(enb1), enw2, row2(enb2), row2(eng),
                    row2(enbb), 5000)
    mea = jnp.pad(mesh_edge_attr, ((0, 0), (0, 16 - mesh_edge_attr.shape[1])))
    emw1p = jnp.pad(emw1, ((0, 16 - emw1.shape[0]), (0, 0)))
    me = _enc_mlp(mea, emw1p, row2(emb1), emw2, row2(emb2), row2(emg),
                  row2(embb), 8000)
    wea = jnp.pad(world_edge_attr, ((0, 0), (0, 16 - world_edge_attr.shape[1])))
    eww1p = jnp.pad(eww1, ((0, 16 - eww1.shape[0]), (0, 0)))
    we = _enc_mlp(wea, eww1p, row2(ewb1), eww2, row2(ewb2), row2(ewg),
                  row2(ewbb), 8000)

    xs = dict(
        wproj=jnp.stack([pmw1[:, L:2 * L], pmw1[:, 2 * L:], pww1[:, L:2 * L],
                         pww1[:, 2 * L:]], axis=1),
        am=pmw1[:, :L], aw=pww1[:, :L],
        pmb1=pmb1[:, None, :], pmb2=pmb2[:, None, :], pmw2=pmw2,
        pmg=pmg[:, None, :], pmbb=pmbb[:, None, :],
        pwb1=pwb1[:, None, :], pwb2=pwb2[:, None, :], pww2=pww2,
        pwg=pwg[:, None, :], pwbb=pwbb[:, None, :],
        pa=pnw1[:, :L], pm=pnw1[:, L:2 * L], pw=pnw1[:, 2 * L:],
        pnb1=pnb1[:, None, :], pnb2=pnb2[:, None, :], pnw2=pnw2,
        png=png[:, None, :], pnbb=pnbb[:, None, :],
    )

    def step(carry, w):
        node, me, we = carry
        tbl = _proj(node, w['wproj'], 5000)
        gsum = _sc_gather_add(tbl, idx_src, idx_dst)
        me = _edge_mlp(me, gsum, 0, w['am'], w['pmb1'], w['pmw2'],
                       w['pmb2'], w['pmg'], w['pmbb'], 8000)
        we = _edge_mlp(we, gsum, E, w['aw'], w['pwb1'],
                       w['pww2'], w['pwb2'], w['pwg'], w['pwbb'], 8000)
        magg = _sc_scatter(me, de32)
        wagg = _sc_scatter(we, wr32)
        node = _node_mlp(node, magg, wagg, w['pa'], w['pm'], w['pw'],
                         w['pnb1'], w['pnw2'], w['pnb2'], w['png'],
                         w['pnbb'], 5000)
        return (node, me, we), None

    (node, me, we), _ = lax.scan(step, (node, me, we), xs)

    dw2p = jnp.pad(dw2, ((0, 0), (0, L - dw2.shape[1])))
    db2p = jnp.pad(db2, ((0, L - db2.shape[0]),)).reshape(1, L)
    out = _decoder(node, dw1, row2(db1), dw2p, db2p, 5000)
    return out[:, :dw2.shape[1]]


# speedup vs baseline: 1.1980x; 1.0153x over previous
import functools

import jax
import jax.numpy as jnp
from jax import lax
from jax.experimental import pallas as pl
from jax.experimental.pallas import tpu as pltpu
from jax.experimental.pallas import tpu_sc as plsc

N = 10000
E = 160000
WE = 32000
L = 128
S = 15

NC = 2
NS = 16
NW = NC * NS


def _ln_rows(y, g, bb):
    m = jnp.mean(y, axis=-1, keepdims=True)
    v = jnp.mean((y - m) ** 2, axis=-1, keepdims=True)
    return (y - m) * lax.rsqrt(v + 1e-5) * g + bb


def _enc_body(x_ref, w1_ref, b1_ref, w2_ref, b2_ref, g_ref, bb_ref, o_ref):
    x = x_ref[...]
    h = jnp.maximum(
        jnp.dot(x, w1_ref[...], preferred_element_type=jnp.float32) + b1_ref[...], 0.0)
    y = jnp.dot(h, w2_ref[...], preferred_element_type=jnp.float32) + b2_ref[...]
    o_ref[...] = _ln_rows(y, g_ref[...], bb_ref[...])


def _enc_mlp(x, w1, b1, w2, b2, g, bb, br):
    r, k = x.shape
    full = lambda *s: pl.BlockSpec(s, lambda i: tuple(0 for _ in s))
    return pl.pallas_call(
        _enc_body,
        grid=(r // br,),
        in_specs=[
            pl.BlockSpec((br, k), lambda i: (i, 0)),
            full(k, L), full(1, L), full(L, L), full(1, L), full(1, L), full(1, L),
        ],
        out_specs=pl.BlockSpec((br, L), lambda i: (i, 0)),
        out_shape=jax.ShapeDtypeStruct((r, L), jnp.float32),
    )(x, w1, b1, w2, b2, g, bb)


def _pack_bf16_pair(lo, hi):
    ulo = lax.bitcast_convert_type(lo, jnp.uint32)
    uhi = lax.bitcast_convert_type(hi, jnp.uint32)
    rlo = (ulo + jnp.uint32(0x7FFF) + ((ulo >> 16) & jnp.uint32(1))) >> 16
    rhi = ((uhi + jnp.uint32(0x7FFF) + ((uhi >> 16) & jnp.uint32(1)))
           & jnp.uint32(0xFFFF0000))
    return rlo | rhi


def _unpack_bf16_pair(u):
    lo = lax.bitcast_convert_type(u << 16, jnp.float32)
    hi = lax.bitcast_convert_type(u & jnp.uint32(0xFFFF0000), jnp.float32)
    return jnp.concatenate([lo, hi], axis=-1)


def _proj_body(x_ref, w_ref, o_ref):
    o_ref[...] = jnp.dot(x_ref[...], w_ref[0],
                         preferred_element_type=jnp.float32)


def _proj(node, wstack, bn):
    nb = N // bn
    return pl.pallas_call(
        _proj_body,
        grid=(4, nb),
        in_specs=[
            pl.BlockSpec((bn, L), lambda t, i: (i, 0)),
            pl.BlockSpec((1, L, L), lambda t, i: (t, 0, 0)),
        ],
        out_specs=pl.BlockSpec((bn, L), lambda t, i: (t * nb + i, 0)),
        out_shape=jax.ShapeDtypeStruct((4 * N, L), jnp.float32),
    )(node, wstack)


def _edge_body(x_ref, g12_ref, a_ref, b1_ref, w2_ref, b2_ref,
               g_ref, bb_ref, o_ref):
    x = x_ref[...]
    t = jnp.dot(x, a_ref[...], preferred_element_type=jnp.float32)
    h = jnp.maximum(t + g12_ref[...] + b1_ref[...], 0.0)
    y = jnp.dot(h, w2_ref[...],
                preferred_element_type=jnp.float32) + b2_ref[...]
    o_ref[...] = x + _ln_rows(y, g_ref[...], bb_ref[...])


def _edge_mlp(x, gsum, off, a, b1, w2, b2, g, bb, be):
    r = x.shape[0]
    full = lambda *s: pl.BlockSpec(s, lambda i: tuple(0 for _ in s))
    o1 = off // be
    return pl.pallas_call(
        _edge_body,
        grid=(r // be,),
        in_specs=[
            pl.BlockSpec((be, L), lambda i: (i, 0)),
            pl.BlockSpec((be, L), lambda i, o=o1: (o + i, 0)),
            full(L, L), full(1, L), full(L, L), full(1, L), full(1, L), full(1, L),
        ],
        out_specs=pl.BlockSpec((be, L), lambda i: (i, 0)),
        out_shape=jax.ShapeDtypeStruct((r, L), jnp.float32),
    )(x, gsum, a, b1, w2, b2, g, bb)


def _node_body(x_ref, m0_ref, m1_ref, w0_ref, w1_ref, pa_ref, pm_ref, pw_ref,
               b1_ref, w2_ref, b2_ref, g_ref, bb_ref, o_ref):
    x = x_ref[...]
    mg = m0_ref[0] + m1_ref[0]
    wg = w0_ref[0] + w1_ref[0]
    t = (jnp.dot(x, pa_ref[...], preferred_element_type=jnp.float32)
         + jnp.dot(mg, pm_ref[...], preferred_element_type=jnp.float32)
         + jnp.dot(wg, pw_ref[...], preferred_element_type=jnp.float32)
         + b1_ref[...])
    h = jnp.maximum(t, 0.0)
    y = jnp.dot(h, w2_ref[...],
                preferred_element_type=jnp.float32) + b2_ref[...]
    o_ref[...] = x + _ln_rows(y, g_ref[...], bb_ref[...])


def _node_mlp(node, magg, wagg, pa, pm, pw, b1, w2, b2, g, bb, bn):
    full = lambda *s: pl.BlockSpec(s, lambda i: tuple(0 for _ in s))
    part = lambda c: pl.BlockSpec((1, bn, L), lambda i, c=c: (c, i, 0))
    return pl.pallas_call(
        _node_body,
        grid=(N // bn,),
        in_specs=[
            pl.BlockSpec((bn, L), lambda i: (i, 0)),
            part(0), part(1), part(0), part(1),
            full(L, L), full(L, L), full(L, L),
            full(1, L), full(L, L), full(1, L), full(1, L), full(1, L),
        ],
        out_specs=pl.BlockSpec((bn, L), lambda i: (i, 0)),
        out_shape=jax.ShapeDtypeStruct((N, L), jnp.float32),
    )(node, magg, magg, wagg, wagg, pa, pm, pw, b1, w2, b2, g, bb)


def _dec_body(x_ref, w1_ref, b1_ref, w2_ref, b2_ref, o_ref):
    h = jnp.maximum(
        jnp.dot(x_ref[...], w1_ref[...], preferred_element_type=jnp.float32)
        + b1_ref[...], 0.0)
    o_ref[...] = jnp.dot(h, w2_ref[...],
                         preferred_element_type=jnp.float32) + b2_ref[...]


def _decoder(node, w1, b1, w2p, b2p, bn):
    full = lambda *s: pl.BlockSpec(s, lambda i: tuple(0 for _ in s))
    return pl.pallas_call(
        _dec_body,
        grid=(N // bn,),
        in_specs=[pl.BlockSpec((bn, L), lambda i: (i, 0)),
                  full(L, L), full(1, L), full(L, L), full(1, L)],
        out_specs=pl.BlockSpec((bn, L), lambda i: (i, 0)),
        out_shape=jax.ShapeDtypeStruct((N, L), jnp.float32),
    )(node, w1, b1, w2p, b2p)


def _sc_mesh():
    return plsc.VectorSubcoreMesh(core_axis_name="c", subcore_axis_name="s",
                                  num_cores=NC, num_subcores=NS)

_G_CH = 200


def _sc_gather_add(table, idx1, idx2):
    r = idx1.shape[0]
    per_w = r // NW
    nch = per_w // _G_CH
    assert nch % 2 == 0

    @functools.partial(
        pl.kernel,
        out_type=jax.ShapeDtypeStruct((r, L), jnp.float32),
        mesh=_sc_mesh(),
        scratch_types=[
            pltpu.VMEM((per_w,), jnp.int32),
            pltpu.VMEM((per_w,), jnp.int32),
            pltpu.VMEM((_G_CH, L), jnp.float32),
            pltpu.VMEM((_G_CH, L), jnp.float32),
            pltpu.VMEM((_G_CH, L), jnp.float32),
            pltpu.VMEM((_G_CH, L), jnp.float32),
            pltpu.SemaphoreType.DMA,
            pltpu.SemaphoreType.DMA,
            pltpu.SemaphoreType.DMA,
            pltpu.SemaphoreType.DMA,
            pltpu.SemaphoreType.DMA,
        ],
    )
    def k(table_hbm, idx1_hbm, idx2_hbm, out_hbm, i1_v, i2_v,
          r1a, r2a, r1b, r2b, s1a, s2a, s1b, s2b, so):
        wid = lax.axis_index("s") * NC + lax.axis_index("c")
        base = wid * per_w
        pltpu.sync_copy(idx1_hbm.at[pl.ds(base, per_w)], i1_v)
        pltpu.sync_copy(idx2_hbm.at[pl.ds(base, per_w)], i2_v)

        def addloop(ra, rb):
            def arow(j, _):
                for c in range(L // 16):
                    sl = pl.ds(c * 16, 16)
                    ra[j, sl] = ra[j, sl] + rb[j, sl]
                return 0
            lax.fori_loop(0, _G_CH, arow, 0, unroll=False)

        def outer(j, _):
            o0 = j * 2 * _G_CH
            d1a = pltpu.async_copy(
                table_hbm.at[i1_v.at[pl.ds(o0, _G_CH)]], r1a, s1a)
            d2a = pltpu.async_copy(
                table_hbm.at[i2_v.at[pl.ds(o0, _G_CH)]], r2a, s2a)
            d1b = pltpu.async_copy(
                table_hbm.at[i1_v.at[pl.ds(o0 + _G_CH, _G_CH)]], r1b, s1b)
            d2b = pltpu.async_copy(
                table_hbm.at[i2_v.at[pl.ds(o0 + _G_CH, _G_CH)]], r2b, s2b)
            d1a.wait()
            d2a.wait()
            addloop(r1a, r2a)
            wa = pltpu.async_copy(r1a, out_hbm.at[pl.ds(base + o0, _G_CH)], so)
            d1b.wait()
            d2b.wait()
            addloop(r1b, r2b)
            wb = pltpu.async_copy(
                r1b, out_hbm.at[pl.ds(base + o0 + _G_CH, _G_CH)], so)
            wa.wait()
            wb.wait()
            return 0

        lax.fori_loop(0, nch // 2, outer, 0, unroll=False)

    return k(table, idx1, idx2)


_S_CH = 200
NP = 10240
_ZROWS = 128


def _sc_scatter(vals, dst):
    r = vals.shape[0]
    per_c = r // NC
    per_w = r // NW
    nch = per_w // _S_CH

    @functools.partial(
        pl.kernel,
        out_type=jax.ShapeDtypeStruct((NC, NP, L), jnp.float32),
        mesh=_sc_mesh(),
        scratch_types=[
            pltpu.VMEM((_S_CH,), jnp.int32),
            pltpu.VMEM((_S_CH, L), jnp.float32),
            pltpu.VMEM((_ZROWS, L), jnp.float32),
            pltpu.VMEM_SHARED((NP, L), jnp.float32),
        ],
    )
    def k(vals_hbm, dst_hbm, out_hbm, idx_v, rows_v, zbuf, acc):
        c = lax.axis_index("c")
        s = lax.axis_index("s")
        z16 = jnp.zeros((16,), jnp.float32)

        def zrow(i, _):
            for j in range(L // 16):
                zbuf[i, pl.ds(j * 16, 16)] = z16
            return 0

        lax.fori_loop(0, _ZROWS, zrow, 0, unroll=False)
        for j in range(NP // (NS * _ZROWS)):
            pltpu.sync_copy(
                zbuf, acc.at[pl.ds(s * (NP // NS) + j * _ZROWS, _ZROWS)])
        plsc.subcore_barrier()

        def body(i, _):
            b = c * per_c + s * per_w + i * _S_CH
            pltpu.sync_copy(dst_hbm.at[pl.ds(b, _S_CH)], idx_v)
            pltpu.sync_copy(vals_hbm.at[pl.ds(b, _S_CH)], rows_v)
            pltpu.sync_copy(rows_v, acc.at[idx_v], add=True)
            return 0

        lax.fori_loop(0, nch, body, 0, unroll=False)
        plsc.subcore_barrier()
        pltpu.sync_copy(acc.at[pl.ds(s * (NP // NS), NP // NS)],
                        out_hbm.at[c, pl.ds(s * (NP // NS), NP // NS)])

    return k(vals, dst)


def kernel(node_feat, edge_index, mesh_edge_attr, world_edge_index,
           world_edge_attr, enw1, enb1, enw2, enb2, eng, enbb, emw1, emb1,
           emw2, emb2, emg, embb, eww1, ewb1, eww2, ewb2, ewg, ewbb, pmw1,
           pmb1, pmw2, pmb2, pmg, pmbb, pww1, pwb1, pww2, pwb2, pwg, pwbb,
           pnw1, pnb1, pnw2, pnb2, png, pnbb, dw1, db1, dw2, db2):
    se, de = edge_index[0], edge_index[1]
    ws, wr = world_edge_index[0], world_edge_index[1]
    idx_src = jnp.concatenate([se, ws + 2 * N]).astype(jnp.int32)
    idx_dst = jnp.concatenate([de + N, wr + 3 * N]).astype(jnp.int32)
    de32 = de.astype(jnp.int32)
    wr32 = wr.astype(jnp.int32)

    row2 = lambda a: a.reshape(1, L)

    nf = jnp.pad(node_feat, ((0, 0), (0, 16 - node_feat.shape[1])))
    enw1p = jnp.pad(enw1, ((0, 16 - enw1.shape[0]), (0, 0)))
    node = _enc_mlp(nf, enw1p, row2(enb1), enw2, row2(enb2), row2(eng),
                    row2(enbb), 10000)
    mea = jnp.pad(mesh_edge_attr, ((0, 0), (0, 16 - mesh_edge_attr.shape[1])))
    emw1p = jnp.pad(emw1, ((0, 16 - emw1.shape[0]), (0, 0)))
    me = _enc_mlp(mea, emw1p, row2(emb1), emw2, row2(emb2), row2(emg),
                  row2(embb), 16000)
    wea = jnp.pad(world_edge_attr, ((0, 0), (0, 16 - world_edge_attr.shape[1])))
    eww1p = jnp.pad(eww1, ((0, 16 - eww1.shape[0]), (0, 0)))
    we = _enc_mlp(wea, eww1p, row2(ewb1), eww2, row2(ewb2), row2(ewg),
                  row2(ewbb), 16000)

    xs = dict(
        wproj=jnp.stack([pmw1[:, L:2 * L], pmw1[:, 2 * L:], pww1[:, L:2 * L],
                         pww1[:, 2 * L:]], axis=1),
        am=pmw1[:, :L], aw=pww1[:, :L],
        pmb1=pmb1[:, None, :], pmb2=pmb2[:, None, :], pmw2=pmw2,
        pmg=pmg[:, None, :], pmbb=pmbb[:, None, :],
        pwb1=pwb1[:, None, :], pwb2=pwb2[:, None, :], pww2=pww2,
        pwg=pwg[:, None, :], pwbb=pwbb[:, None, :],
        pa=pnw1[:, :L], pm=pnw1[:, L:2 * L], pw=pnw1[:, 2 * L:],
        pnb1=pnb1[:, None, :], pnb2=pnb2[:, None, :], pnw2=pnw2,
        png=png[:, None, :], pnbb=pnbb[:, None, :],
    )

    def step(carry, w):
        node, me, we = carry
        tbl = _proj(node, w['wproj'], 10000)
        gsum = _sc_gather_add(tbl, idx_src, idx_dst)
        me = _edge_mlp(me, gsum, 0, w['am'], w['pmb1'], w['pmw2'],
                       w['pmb2'], w['pmg'], w['pmbb'], 8000)
        we = _edge_mlp(we, gsum, E, w['aw'], w['pwb1'],
                       w['pww2'], w['pwb2'], w['pwg'], w['pwbb'], 8000)
        magg = _sc_scatter(me, de32)
        wagg = _sc_scatter(we, wr32)
        node = _node_mlp(node, magg, wagg, w['pa'], w['pm'], w['pw'],
                         w['pnb1'], w['pnw2'], w['pnb2'], w['png'],
                         w['pnbb'], 5000)
        return (node, me, we), None

    (node, me, we), _ = lax.scan(step, (node, me, we), xs)

    dw2p = jnp.pad(dw2, ((0, 0), (0, L - dw2.shape[1])))
    db2p = jnp.pad(db2, ((0, L - db2.shape[0]),)).reshape(1, L)
    out = _decoder(node, dw1, row2(db1), dw2p, db2p, 10000)
    return out[:, :dw2.shape[1]]
